# baseline (speedup 1.0000x reference)
import functools

import jax
import jax.numpy as jnp
from jax import lax
from jax.experimental import pallas as pl
from jax.experimental.pallas import tpu as pltpu
from jax.experimental.pallas import tpu_sc as plsc

BATCH = 4096
MAXLEN = 200
EMBED = 64
LANES = 16
GROUPS = EMBED // LANES

NUM_CORES = 2
NUM_SUBCORES = 16
NUM_WORKERS = NUM_CORES * NUM_SUBCORES
ROWS_PER_WORKER = BATCH // NUM_WORKERS
CHUNK_ROWS = 2
CHUNK_TOK = CHUNK_ROWS * MAXLEN
NUM_CHUNKS = ROWS_PER_WORKER // CHUNK_ROWS


def _body(x_hbm, tok_hbm, pos_hbm, out_hbm, idx_v, rows_v, pos_v, sem):
    wid = lax.axis_index("s") * NUM_CORES + lax.axis_index("c")
    worker_base = wid * ROWS_PER_WORKER * MAXLEN

    pltpu.sync_copy(pos_hbm, pos_v)

    def chunk_body(c, _):
        base = worker_base + c * CHUNK_TOK
        pltpu.sync_copy(x_hbm.at[pl.ds(base, CHUNK_TOK)], idx_v)
        pltpu.async_copy(tok_hbm.at[idx_v], rows_v, sem).wait()

        def add_row(j, _):
            for rr in range(CHUNK_ROWS):
                for g in range(GROUPS):
                    sl = pl.ds(g * LANES, LANES)
                    rows_v[rr * MAXLEN + j, sl] = (
                        rows_v[rr * MAXLEN + j, sl] + pos_v[j, sl]
                    )
            return 0

        lax.fori_loop(0, MAXLEN, add_row, 0, unroll=2)
        pltpu.sync_copy(rows_v, out_hbm.at[pl.ds(base, CHUNK_TOK)])
        return 0

    lax.fori_loop(0, NUM_CHUNKS, chunk_body, 0)


@jax.jit
def _embed(x_flat, token_table, pos_table):
    mesh = plsc.VectorSubcoreMesh(core_axis_name="c", subcore_axis_name="s")
    k = functools.partial(
        pl.kernel,
        mesh=mesh,
        out_type=jax.ShapeDtypeStruct((BATCH * MAXLEN, EMBED), jnp.float32),
        scratch_types=[
            pltpu.VMEM((CHUNK_TOK,), jnp.int32),
            pltpu.VMEM((CHUNK_TOK, EMBED), jnp.float32),
            pltpu.VMEM((MAXLEN, EMBED), jnp.float32),
            pltpu.SemaphoreType.DMA,
        ],
        compiler_params=pltpu.CompilerParams(use_tc_tiling_on_sc=False),
    )(_body)
    return k(x_flat, token_table, pos_table)


def kernel(x, token_table, pos_table):
    x_flat = x.reshape(-1).astype(jnp.int32)
    out = _embed(x_flat, token_table, pos_table)
    return out.reshape(BATCH, MAXLEN, EMBED)


# speedup vs baseline: 1.1124x; 1.1124x over previous
import functools

import jax
import jax.numpy as jnp
from jax import lax
from jax.experimental import pallas as pl
from jax.experimental.pallas import tpu as pltpu
from jax.experimental.pallas import tpu_sc as plsc

BATCH = 4096
MAXLEN = 200
EMBED = 64
LANES = 16
GROUPS = EMBED // LANES

NUM_CORES = 2
NUM_SUBCORES = 16
NUM_WORKERS = NUM_CORES * NUM_SUBCORES
ROWS_PER_WORKER = BATCH // NUM_WORKERS
CHUNK_ROWS = 2
CHUNK_TOK = CHUNK_ROWS * MAXLEN
NUM_CHUNKS = ROWS_PER_WORKER // CHUNK_ROWS


def _body(x_hbm, tok_hbm, pos_hbm, out_hbm,
          idx0, idx1, rows0, rows1, out0, out1, pos_v,
          gsem, isem, osem):
    idx_v = (idx0, idx1)
    rows_v = (rows0, rows1)
    out_v = (out0, out1)
    wid = lax.axis_index("s") * NUM_CORES + lax.axis_index("c")
    worker_base = wid * ROWS_PER_WORKER * MAXLEN

    pltpu.sync_copy(pos_hbm, pos_v)

    def tok_base(c):
        return worker_base + c * CHUNK_TOK

    def idx_start(c, p):
        return pltpu.async_copy(
            x_hbm.at[pl.ds(tok_base(c), CHUNK_TOK)], idx_v[p], isem.at[p])

    def gather_start(p):
        return pltpu.async_copy(tok_hbm.at[idx_v[p]], rows_v[p], gsem.at[p])

    def gather_wait(p):
        pltpu.make_async_copy(tok_hbm.at[idx_v[p]], rows_v[p], gsem.at[p]).wait()

    def wb_start(c, p):
        return pltpu.async_copy(
            out_v[p], out_hbm.at[pl.ds(tok_base(c), CHUNK_TOK)], osem.at[p])

    def wb_wait(c, p):
        pltpu.make_async_copy(
            out_v[p], out_hbm.at[pl.ds(tok_base(c), CHUNK_TOK)], osem.at[p]).wait()

    def add_pos(p):
        def add_body(j, _):
            for g in range(GROUPS):
                sl = pl.ds(g * LANES, LANES)
                pv = pos_v[j, sl]
                for rr in range(CHUNK_ROWS):
                    out_v[p][rr * MAXLEN + j, sl] = rows_v[p][rr * MAXLEN + j, sl] + pv
            return 0
        lax.fori_loop(0, MAXLEN, add_body, 0, unroll=2)

    for c in range(2):
        idx_start(c, c).wait()
        gather_start(c)

    for c in range(2):
        p = c
        gather_wait(p)
        idx_start(c + 2, p)
        add_pos(p)
        wb_start(c, p)
        pltpu.make_async_copy(
            x_hbm.at[pl.ds(tok_base(c + 2), CHUNK_TOK)], idx_v[p], isem.at[p]).wait()
        gather_start(p)

    def pair_body(gg, _):
        for b in range(2):
            c = 2 + 2 * gg + b
            p = b
            gather_wait(p)
            idx_start(c + 2, p)
            wb_wait(c - 2, p)
            add_pos(p)
            wb_start(c, p)
            pltpu.make_async_copy(
                x_hbm.at[pl.ds(tok_base(c + 2), CHUNK_TOK)], idx_v[p], isem.at[p]).wait()
            gather_start(p)
        return 0

    lax.fori_loop(0, (NUM_CHUNKS - 4) // 2, pair_body, 0)

    for c in range(NUM_CHUNKS - 2, NUM_CHUNKS):
        p = c % 2
        gather_wait(p)
        wb_wait(c - 2, p)
        add_pos(p)
        wb_start(c, p)

    for c in range(NUM_CHUNKS - 2, NUM_CHUNKS):
        wb_wait(c, c % 2)


@jax.jit
def _embed(x_flat, token_table, pos_table):
    mesh = plsc.VectorSubcoreMesh(core_axis_name="c", subcore_axis_name="s")
    k = functools.partial(
        pl.kernel,
        mesh=mesh,
        out_type=jax.ShapeDtypeStruct((BATCH * MAXLEN, EMBED), jnp.float32),
        scratch_types=[
            pltpu.VMEM((CHUNK_TOK,), jnp.int32),
            pltpu.VMEM((CHUNK_TOK,), jnp.int32),
            pltpu.VMEM((CHUNK_TOK, EMBED), jnp.float32),
            pltpu.VMEM((CHUNK_TOK, EMBED), jnp.float32),
            pltpu.VMEM((CHUNK_TOK, EMBED), jnp.float32),
            pltpu.VMEM((CHUNK_TOK, EMBED), jnp.float32),
            pltpu.VMEM((MAXLEN, EMBED), jnp.float32),
            pltpu.SemaphoreType.DMA((2,)),
            pltpu.SemaphoreType.DMA((2,)),
            pltpu.SemaphoreType.DMA((2,)),
        ],
        compiler_params=pltpu.CompilerParams(use_tc_tiling_on_sc=False),
    )(_body)
    return k(x_flat, token_table, pos_table)


def kernel(x, token_table, pos_table):
    x_flat = x.reshape(-1).astype(jnp.int32)
    out = _embed(x_flat, token_table, pos_table)
    return out.reshape(BATCH, MAXLEN, EMBED)
